# baseline (speedup 1.0000x reference)
import functools

import jax
import jax.numpy as jnp
from jax.experimental import pallas as pl
from jax.experimental.pallas import tpu as pltpu

_H = 8
_DH = 32
_KCB = 1024
_LIN = 64
_D = 256


def _mm(a, b):
    return jnp.dot(a.astype(jnp.bfloat16), b.astype(jnp.bfloat16),
                   preferred_element_type=jnp.float32)


def _stride8(x):
    acc = x[:, 0:8]
    for i in range(1, x.shape[1] // 8):
        acc = acc + x[:, 8 * i:8 * (i + 1)]
    return acc


def _rowsum(x):
    w = x.shape[1]
    if w > 128 and w % 128 == 0:
        acc = x[:, 0:128]
        for i in range(1, w // 128):
            acc = acc + x[:, 128 * i:128 * (i + 1)]
        y = _stride8(acc)
    else:
        y = _stride8(x)
    y = y[:, 0:4] + y[:, 4:8]
    y = y[:, 0:2] + y[:, 2:4]
    return y[:, 0:1] + y[:, 1:2]


def _softmax(x):
    m = jnp.max(x, -1, keepdims=True)
    u = jnp.exp(x - m)
    return u * (1.0 / _rowsum(u))


def _ln(x, g, b, eps=1e-5):
    n = jnp.float32(x.shape[-1])
    m = _rowsum(x) / n
    d = x - m
    v = _rowsum(d * d) / n
    return d / jnp.sqrt(v + eps) * g + b


def _mha(q_in, k_in, v_in, p):
    q = _mm(q_in, p['Wq']) + p['bq']
    k = _mm(k_in, p['Wk']) + p['bk']
    v = _mm(v_in, p['Wv']) + p['bv']
    scale = jnp.sqrt(jnp.float32(_DH))
    rows = []
    for h in range(_H):
        sl = slice(h * _DH, (h + 1) * _DH)
        qh, kh, vh = q[:, sl], k[:, sl], v[:, sl]
        s = jax.lax.dot_general(
            qh.astype(jnp.bfloat16), kh.astype(jnp.bfloat16),
            (((1,), (1,)), ((), ())),
            preferred_element_type=jnp.float32) / scale
        a = _softmax(s)
        rows.append(jax.lax.dot_general(
            vh.astype(jnp.bfloat16), a.astype(jnp.bfloat16),
            (((0,), (1,)), ((), ())),
            preferred_element_type=jnp.float32))
    o = jnp.concatenate(rows, axis=0).T
    return _mm(o, p['Wo']) + p['bo']


def _ffn(x, p):
    h = jax.nn.relu(_mm(x, p['W1']) + p['b1'])
    return _mm(h, p['W2']) + p['b2']


def _enc_layer(x, p):
    x = _ln(x + _mha(x, x, x, p['sa']), p['ln1_g'], p['ln1_b'])
    x = _ln(x + _ffn(x, p['ff']), p['ln2_g'], p['ln2_b'])
    return x


def _dec_layer(t, mem, p):
    t = _ln(t + _mha(t, t, t, p['sa']), p['ln1_g'], p['ln1_b'])
    t = _ln(t + _mha(t, mem, mem, p['ca']), p['ln2_g'], p['ln2_b'])
    t = _ln(t + _ffn(t, p['ff']), p['ln3_g'], p['ln3_b'])
    return t


def _transformer(src, tgt, p):
    mem = src
    for lp in p['enc_layers']:
        mem = _enc_layer(mem, lp)
    mem = _ln(mem, p['enc_norm_g'], p['enc_norm_b'])
    out = tgt
    for lp in p['dec_layers']:
        out = _dec_layer(out, mem, lp)
    out = _ln(out, p['dec_norm_g'], p['dec_norm_b'])
    return out


def _mm320(x, w, wtT):
    head = _mm(x, w)[:, :256]
    tail = jax.lax.dot_general(
        wtT.astype(jnp.bfloat16), x.astype(jnp.bfloat16),
        (((1,), (1,)), ((), ())), preferred_element_type=jnp.float32).T
    return jnp.concatenate([head, tail], axis=1)


def _mlp(x, p):
    x = jax.nn.silu(_ln(_mm320(x, p['W1'], p['W1tT']) + p['b1'],
                        p['ln1_g'], p['ln1_b']))
    x = jax.nn.silu(_ln(_mm(x, p['W2']) + p['b2'], p['ln2_g'], p['ln2_b']))
    return x


def _body(n_leaves, treedef, x_ref, *refs):
    p = jax.tree_util.tree_unflatten(
        treedef, [r[...] for r in refs[:n_leaves]])
    z_ref, zq_ref, c_ref, oh_ref, rec_ref = refs[n_leaves:]

    xb = x_ref[0]
    h = _mlp(xb, p['enc_mlp'])
    z = _transformer(h, p['input_state'], p['tf_enc'])

    z2 = _rowsum(z * z)
    d2 = z2 - 2.0 * _mm(z, p['cb_t']) + p['cb2']
    mind = jnp.min(d2, axis=-1, keepdims=True)
    iota = jax.lax.broadcasted_iota(jnp.int32, (_LIN, _KCB), 1)
    c2 = jnp.min(jnp.where(d2 <= mind, iota, _KCB), axis=-1,
                 keepdims=True)
    oh = (iota == c2).astype(jnp.float32)
    z_q = jnp.dot(oh, p['codebook'], preferred_element_type=jnp.float32,
                  precision=jax.lax.Precision.HIGHEST)

    rec = _transformer(z_q, p['output_state'], p['tf_dec'])
    rec = _mlp(rec, p['dec_mlp'])

    z_ref[0] = z
    zq_ref[0] = z_q
    c_ref[0] = c2
    oh_ref[0] = oh
    rec_ref[0] = rec


def _zero_map(nd, b):
    return (0,) * nd


def _batch_map(b):
    return (b, 0, 0)


def kernel(x, params):
    B, S, in_dim = x.shape
    out_dim = params['dec_mlp']['W2'].shape[1]
    p2 = jax.tree.map(lambda a: a.reshape(1, -1) if a.ndim == 1 else a, params)
    p2['cb_t'] = params['codebook'].T
    p2['cb2'] = jnp.sum(params['codebook'] ** 2, -1)[None, :]
    p2['enc_mlp']['W1tT'] = params['enc_mlp']['W1'].T[256:, :]
    p2['dec_mlp']['W1tT'] = params['dec_mlp']['W1'].T[256:, :]
    leaves, treedef = jax.tree_util.tree_flatten(p2)

    out_shape = (
        jax.ShapeDtypeStruct((B, _LIN, _D), jnp.float32),
        jax.ShapeDtypeStruct((B, _LIN, _D), jnp.float32),
        jax.ShapeDtypeStruct((B, _LIN, 1), jnp.int32),
        jax.ShapeDtypeStruct((B, _LIN, _KCB), jnp.float32),
        jax.ShapeDtypeStruct((B, S, out_dim), jnp.float32),
    )
    in_specs = [pl.BlockSpec((1, S, in_dim), _batch_map)]
    in_specs += [pl.BlockSpec(l.shape, functools.partial(_zero_map, l.ndim))
                 for l in leaves]
    out_specs = [
        pl.BlockSpec((1, _LIN, _D), _batch_map),
        pl.BlockSpec((1, _LIN, _D), _batch_map),
        pl.BlockSpec((1, _LIN, 1), _batch_map),
        pl.BlockSpec((1, _LIN, _KCB), _batch_map),
        pl.BlockSpec((1, S, out_dim), _batch_map),
    ]
    z, z_q, c3, oh, rec = pl.pallas_call(
        functools.partial(_body, len(leaves), treedef),
        grid=(B,),
        in_specs=in_specs,
        out_specs=out_specs,
        out_shape=out_shape,
        compiler_params=pltpu.CompilerParams(
            dimension_semantics=("arbitrary",)),
    )(x, *leaves)
    return (z, z_q, c3[:, :, 0], oh, rec)
